# Initial kernel scaffold; baseline (speedup 1.0000x reference)
import jax
import jax.numpy as jnp
from jax.experimental import pallas as pl


def kernel(W, deltas, x, idx):
    raise NotImplementedError("write your pallas kernel here")


# speedup vs baseline: 3.5694x; 3.5694x over previous
import functools

import jax
import jax.numpy as jnp
from jax import lax
from jax.experimental import pallas as pl
from jax.experimental.pallas import tpu as pltpu
from jax.experimental.pallas import tpu_sc as plsc

M, D, B = 8192, 8192, 32
NNZ = 671088
NW = 32
GS = 128
CH = -(-(-(-NNZ // NW)) // GS) * GS
NNZ_PAD = NW * CH
NSUB = CH // GS
BN = 256
RPT = D // 16


def _sc_sparse_partials(xT, idx_p, dlt_p, zeros):
    mesh = plsc.VectorSubcoreMesh(core_axis_name="c", subcore_axis_name="s")

    @functools.partial(
        pl.kernel,
        mesh=mesh,
        compiler_params=pltpu.CompilerParams(use_tc_tiling_on_sc=False),
        out_type=jax.ShapeDtypeStruct((2 * D, B), jnp.float32),
        scratch_types=[
            pltpu.VMEM((GS,), jnp.int32),
            pltpu.VMEM((GS,), jnp.int32),
            pltpu.VMEM((GS,), jnp.int32),
            pltpu.VMEM((GS,), jnp.float32),
            pltpu.VMEM((GS, B), jnp.float32),
            pltpu.VMEM_SHARED((D, B), jnp.float32),
            pltpu.SemaphoreType.DMA,
        ],
    )
    def k(xT_hbm, idx_hbm, dlt_hbm, z_hbm, out_hbm,
          idxb, rb, cb, db, rows, acc, sem):
        cid = lax.axis_index("c")
        sid = lax.axis_index("s")
        wid = cid * 16 + sid
        pltpu.sync_copy(z_hbm.at[pl.ds(sid * RPT, RPT)],
                        acc.at[pl.ds(sid * RPT, RPT)])
        plsc.subcore_barrier()
        base = wid * CH

        def body(j, carry):
            off = base + j * GS
            pltpu.sync_copy(idx_hbm.at[pl.ds(off, GS)], idxb)
            pltpu.sync_copy(dlt_hbm.at[pl.ds(off, GS)], db)
            for t in range(GS // 16):
                sl = pl.ds(t * 16, 16)
                v = idxb[sl]
                rb[sl] = lax.shift_right_logical(v, 13)
                cb[sl] = lax.bitwise_and(v, D - 1)
            pltpu.async_copy(xT_hbm.at[rb], rows, sem).wait()
            for g in range(GS // 16):
                dvec = db[pl.ds(g * 16, 16)]
                for l in range(16):
                    dscal = dvec[l]
                    i = g * 16 + l
                    for h in range(B // 16):
                        sl = pl.ds(h * 16, 16)
                        rows[i, sl] = rows[i, sl] * dscal
            pltpu.sync_copy(rows, acc.at[cb], add=True)
            return carry

        lax.fori_loop(0, NSUB, body, 0)
        plsc.subcore_barrier()
        pltpu.sync_copy(acc.at[pl.ds(sid * RPT, RPT)],
                        out_hbm.at[pl.ds(cid * D + sid * RPT, RPT)])

    return k(xT, idx_p, dlt_p, zeros)


def _mm_body(x_ref, w_ref, p_ref, o_ref):
    d = jnp.dot(x_ref[...], w_ref[...], preferred_element_type=jnp.float32)
    p = p_ref[0] + p_ref[1]
    o_ref[...] = d + p.T


def _matmul_add(x, W, P3):
    grid = (D // BN,)
    return pl.pallas_call(
        _mm_body,
        grid=grid,
        in_specs=[
            pl.BlockSpec((B, M), lambda j: (0, 0)),
            pl.BlockSpec((M, BN), lambda j: (0, j)),
            pl.BlockSpec((2, BN, B), lambda j: (0, j, 0)),
        ],
        out_specs=pl.BlockSpec((B, BN), lambda j: (0, j)),
        out_shape=jax.ShapeDtypeStruct((B, D), jnp.float32),
    )(x, W, P3)


def kernel(W, deltas, x, idx):
    pad = NNZ_PAD - NNZ
    idx_p = jnp.concatenate([idx, jnp.zeros((pad,), jnp.int32)])
    dlt_p = jnp.concatenate([deltas, jnp.zeros((pad,), jnp.float32)])
    xT = x.T
    zeros = jnp.zeros((D, B), jnp.float32)
    P = _sc_sparse_partials(xT, idx_p, dlt_p, zeros)
    P3 = P.reshape(2, D, B)
    return _matmul_add(x, W, P3)


# speedup vs baseline: 7.8259x; 2.1925x over previous
import functools

import jax
import jax.numpy as jnp
from jax import lax
from jax.experimental import pallas as pl
from jax.experimental.pallas import tpu as pltpu
from jax.experimental.pallas import tpu_sc as plsc

M, D, B = 8192, 8192, 32
NNZ = 671088
NW = 32
GS = 128
CH = -(-(-(-NNZ // NW)) // GS) * GS
NNZ_PAD = NW * CH
NSUB = CH // GS
NBUF = 4
BN = 256
RPT = D // 16


def _sc_sparse_partials(xT, idx_p, dlt_p, zeros):
    mesh = plsc.VectorSubcoreMesh(core_axis_name="c", subcore_axis_name="s")

    @functools.partial(
        pl.kernel,
        mesh=mesh,
        compiler_params=pltpu.CompilerParams(use_tc_tiling_on_sc=False),
        out_type=jax.ShapeDtypeStruct((2 * D, B), jnp.float32),
        scratch_types=[
            pltpu.VMEM((CH,), jnp.int32),
            pltpu.VMEM((NSUB, GS), jnp.int32),
            pltpu.VMEM((CH,), jnp.float32),
            [pltpu.VMEM((GS, B), jnp.float32) for _ in range(NBUF)],
            [pltpu.SemaphoreType.DMA for _ in range(NBUF)],
            [pltpu.SemaphoreType.DMA for _ in range(NBUF)],
            pltpu.VMEM_SHARED((D, B), jnp.float32),
        ],
    )
    def k(xT_hbm, idx_hbm, dlt_hbm, z_hbm, out_hbm,
          ib, cb, db, rows, gsem, ssem, acc):
        cid = lax.axis_index("c")
        sid = lax.axis_index("s")
        wid = cid * 16 + sid
        base = wid * CH
        pltpu.async_copy(idx_hbm.at[pl.ds(base, CH)], ib, gsem[0])
        pltpu.async_copy(dlt_hbm.at[pl.ds(base, CH)], db, gsem[1])
        pltpu.sync_copy(z_hbm.at[pl.ds(sid * RPT, RPT)],
                        acc.at[pl.ds(sid * RPT, RPT)])
        pltpu.make_async_copy(idx_hbm.at[pl.ds(base, CH)], ib, gsem[0]).wait()
        pltpu.make_async_copy(dlt_hbm.at[pl.ds(base, CH)], db, gsem[1]).wait()

        def rc_body(j, carry):
            for t in range(GS // 16):
                sl = pl.ds(j * GS + t * 16, 16)
                v = ib[sl]
                ib[sl] = lax.shift_right_logical(v, 13)
                cb[j, pl.ds(t * 16, 16)] = lax.bitwise_and(v, D - 1)
            return carry

        lax.fori_loop(0, NSUB, rc_body, 0)
        plsc.subcore_barrier()

        def g_start(j, b):
            pltpu.async_copy(xT_hbm.at[ib.at[pl.ds(j * GS, GS)]],
                             rows[b], gsem[b])

        def g_wait(b):
            pltpu.make_async_copy(xT_hbm.at[ib.at[pl.ds(0, GS)]],
                                  rows[b], gsem[b]).wait()

        def s_wait(b):
            pltpu.make_async_copy(rows[b], acc.at[cb.at[0]], ssem[b]).wait()

        g_start(0, 0)

        def body(q, carry):
            for b in range(NBUF):
                j = q * NBUF + b
                bn = (b + 1) % NBUF
                @pl.when(jnp.logical_and(j + 1 >= NBUF, j + 1 < NSUB))
                def _():
                    s_wait(bn)

                @pl.when(j + 1 < NSUB)
                def _():
                    g_start(j + 1, bn)

                g_wait(b)
                r = rows[b]
                for g in range(GS // 16):
                    dvec = db[pl.ds(j * GS + g * 16, 16)]
                    for l in range(16):
                        dscal = dvec[l]
                        i = g * 16 + l
                        r[i, pl.ds(0, 16)] = r[i, pl.ds(0, 16)] * dscal
                        r[i, pl.ds(16, 16)] = r[i, pl.ds(16, 16)] * dscal
                pltpu.async_copy(r, acc.at[cb.at[j]], ssem[b], add=True)
            return carry

        lax.fori_loop(0, NSUB // NBUF, body, 0)
        for b in range(NBUF):
            s_wait(b)
        plsc.subcore_barrier()
        pltpu.sync_copy(acc.at[pl.ds(sid * RPT, RPT)],
                        out_hbm.at[pl.ds(cid * D + sid * RPT, RPT)])

    return k(xT, idx_p, dlt_p, zeros)


def _mm_body(x_ref, w_ref, p_ref, o_ref):
    d = jnp.dot(x_ref[...], w_ref[...], preferred_element_type=jnp.float32)
    p = p_ref[0] + p_ref[1]
    o_ref[...] = d + p.T


def _matmul_add(x, W, P3):
    grid = (D // BN,)
    return pl.pallas_call(
        _mm_body,
        grid=grid,
        in_specs=[
            pl.BlockSpec((B, M), lambda j: (0, 0)),
            pl.BlockSpec((M, BN), lambda j: (0, j)),
            pl.BlockSpec((2, BN, B), lambda j: (0, j, 0)),
        ],
        out_specs=pl.BlockSpec((B, BN), lambda j: (0, j)),
        out_shape=jax.ShapeDtypeStruct((B, D), jnp.float32),
    )(x, W, P3)


def kernel(W, deltas, x, idx):
    pad = NNZ_PAD - NNZ
    idx_p = jnp.concatenate([idx, jnp.zeros((pad,), jnp.int32)])
    dlt_p = jnp.concatenate([deltas, jnp.zeros((pad,), jnp.float32)])
    xT = x.T
    zeros = jnp.zeros((D, B), jnp.float32)
    P = _sc_sparse_partials(xT, idx_p, dlt_p, zeros)
    P3 = P.reshape(2, D, B)
    return _matmul_add(x, W, P3)


# speedup vs baseline: 8.5908x; 1.0977x over previous
import functools

import jax
import jax.numpy as jnp
from jax import lax
from jax.experimental import pallas as pl
from jax.experimental.pallas import tpu as pltpu
from jax.experimental.pallas import tpu_sc as plsc

M, D, B = 8192, 8192, 32
NNZ = 671088
NW = 32
GS = 128
CH = -(-(-(-NNZ // NW)) // GS) * GS
NNZ_PAD = NW * CH
NSUB = CH // GS
NBUF = 4
BN = 256
RPT = D // 16


def _sc_sparse_partials(xT, idx_p, dlt_p, zeros):
    mesh = plsc.VectorSubcoreMesh(core_axis_name="c", subcore_axis_name="s")

    @functools.partial(
        pl.kernel,
        mesh=mesh,
        compiler_params=pltpu.CompilerParams(use_tc_tiling_on_sc=False),
        out_type=jax.ShapeDtypeStruct((2 * D, B), jnp.float32),
        scratch_types=[
            pltpu.VMEM((CH,), jnp.int32),
            pltpu.VMEM((NSUB, GS), jnp.int32),
            pltpu.VMEM((CH,), jnp.float32),
            [pltpu.VMEM((GS, B), jnp.float32) for _ in range(NBUF)],
            [pltpu.SemaphoreType.DMA for _ in range(NBUF)],
            [pltpu.SemaphoreType.DMA for _ in range(NBUF)],
            pltpu.VMEM_SHARED((D, B), jnp.float32),
        ],
    )
    def k(xT_hbm, idx_hbm, dlt_hbm, z_hbm, out_hbm,
          ib, cb, db, rows, gsem, ssem, acc):
        cid = lax.axis_index("c")
        sid = lax.axis_index("s")
        wid = cid * 16 + sid
        base = wid * CH
        pltpu.async_copy(idx_hbm.at[pl.ds(base, CH)], ib, gsem[0])
        pltpu.async_copy(dlt_hbm.at[pl.ds(base, CH)], db, gsem[1])
        pltpu.sync_copy(z_hbm.at[pl.ds(sid * RPT, RPT)],
                        acc.at[pl.ds(sid * RPT, RPT)])
        pltpu.make_async_copy(idx_hbm.at[pl.ds(base, CH)], ib, gsem[0]).wait()
        pltpu.make_async_copy(dlt_hbm.at[pl.ds(base, CH)], db, gsem[1]).wait()

        def rc_body(j, carry):
            for t in range(GS // 16):
                sl = pl.ds(j * GS + t * 16, 16)
                v = ib[sl]
                ib[sl] = lax.shift_right_logical(v, 13)
                cb[j, pl.ds(t * 16, 16)] = lax.bitwise_and(v, D - 1)
            return carry

        lax.fori_loop(0, NSUB, rc_body, 0)
        plsc.subcore_barrier()

        def g_start(j, b):
            pltpu.async_copy(xT_hbm.at[ib.at[pl.ds(j * GS, GS)]],
                             rows[b], gsem[b])

        def g_wait(b):
            pltpu.make_async_copy(xT_hbm.at[ib.at[pl.ds(0, GS)]],
                                  rows[b], gsem[b]).wait()

        def s_wait(b):
            pltpu.make_async_copy(rows[b], acc.at[cb.at[0]], ssem[b]).wait()

        g_start(0, 0)

        def body(q, carry):
            for b in range(NBUF):
                j = q * NBUF + b
                bn = (b + 1) % NBUF
                @pl.when(jnp.logical_and(j + 1 >= NBUF, j + 1 < NSUB))
                def _():
                    s_wait(bn)

                @pl.when(j + 1 < NSUB)
                def _():
                    g_start(j + 1, bn)

                g_wait(b)
                r = rows[b]
                for g in range(GS // 16):
                    dvec = db[pl.ds(j * GS + g * 16, 16)]
                    for l in range(16):
                        dscal = dvec[l]
                        i = g * 16 + l
                        r[i, pl.ds(0, 16)] = r[i, pl.ds(0, 16)] * dscal
                        r[i, pl.ds(16, 16)] = r[i, pl.ds(16, 16)] * dscal
                pltpu.async_copy(r, acc.at[cb.at[j]], ssem[b], add=True)
            return carry

        lax.fori_loop(0, NSUB // NBUF, body, 0)
        for b in range(NBUF):
            s_wait(b)
        plsc.subcore_barrier()
        pltpu.sync_copy(acc.at[pl.ds(sid * RPT, RPT)],
                        out_hbm.at[pl.ds(cid * D + sid * RPT, RPT)])

    return k(xT, idx_p, dlt_p, zeros)


def _mm_body(x_ref, w_ref, o_ref):
    o_ref[...] = jnp.dot(x_ref[...], w_ref[...],
                         preferred_element_type=jnp.float32)


def _matmul(x, W):
    grid = (D // BN,)
    return pl.pallas_call(
        _mm_body,
        grid=grid,
        in_specs=[
            pl.BlockSpec((B, M), lambda j: (0, 0)),
            pl.BlockSpec((M, BN), lambda j: (0, j)),
        ],
        out_specs=pl.BlockSpec((B, BN), lambda j: (0, j)),
        out_shape=jax.ShapeDtypeStruct((B, D), jnp.float32),
    )(x, W)


def _add_body(d_ref, p_ref, o_ref):
    p = p_ref[0] + p_ref[1]
    o_ref[...] = d_ref[...] + p.T


def _add_partials(dense, P3):
    grid = (D // BN,)
    return pl.pallas_call(
        _add_body,
        grid=grid,
        in_specs=[
            pl.BlockSpec((B, BN), lambda j: (0, j)),
            pl.BlockSpec((2, BN, B), lambda j: (0, j, 0)),
        ],
        out_specs=pl.BlockSpec((B, BN), lambda j: (0, j)),
        out_shape=jax.ShapeDtypeStruct((B, D), jnp.float32),
    )(dense, P3)


def kernel(W, deltas, x, idx):
    pad = NNZ_PAD - NNZ
    idx_p = jnp.concatenate([idx, jnp.zeros((pad,), jnp.int32)])
    dlt_p = jnp.concatenate([deltas, jnp.zeros((pad,), jnp.float32)])
    xT = x.T
    zeros = jnp.zeros((D, B), jnp.float32)
    P = _sc_sparse_partials(xT, idx_p, dlt_p, zeros)
    dense = _matmul(x, W)
    P3 = P.reshape(2, D, B)
    return _add_partials(dense, P3)


# speedup vs baseline: 9.9287x; 1.1557x over previous
import functools

import jax
import jax.numpy as jnp
from jax import lax
from jax.experimental import pallas as pl
from jax.experimental.pallas import tpu as pltpu
from jax.experimental.pallas import tpu_sc as plsc

M, D, B = 8192, 8192, 32
NNZ = 671088
NW = 32
GS = 128
CH = -(-(-(-NNZ // NW)) // GS) * GS
NNZ_PAD = NW * CH
NSUB = CH // GS
NBUF = 4
LOOK = 2
BN = 512
RPT = D // 16


def _sc_sparse_partials(xTb, idx_p, dlt_p, zeros):
    mesh = plsc.VectorSubcoreMesh(core_axis_name="c", subcore_axis_name="s")

    @functools.partial(
        pl.kernel,
        mesh=mesh,
        compiler_params=pltpu.CompilerParams(use_tc_tiling_on_sc=False,
                                             needs_layout_passes=False),
        out_type=jax.ShapeDtypeStruct((2 * D, B), jnp.bfloat16),
        scratch_types=[
            pltpu.VMEM((CH,), jnp.int32),
            pltpu.VMEM((NSUB, GS), jnp.int32),
            pltpu.VMEM((CH,), jnp.float32),
            [pltpu.VMEM((GS, B), jnp.bfloat16) for _ in range(NBUF)],
            [pltpu.SemaphoreType.DMA for _ in range(NBUF)],
            [pltpu.SemaphoreType.DMA for _ in range(NBUF)],
            pltpu.VMEM_SHARED((D, B), jnp.bfloat16),
        ],
    )
    def k(xT_hbm, idx_hbm, dlt_hbm, z_hbm, out_hbm,
          ib, cb, db, rows, gsem, ssem, acc):
        cid = lax.axis_index("c")
        sid = lax.axis_index("s")
        wid = cid * 16 + sid
        base = wid * CH
        pltpu.async_copy(idx_hbm.at[pl.ds(base, CH)], ib, gsem[0])
        pltpu.async_copy(dlt_hbm.at[pl.ds(base, CH)], db, gsem[1])
        pltpu.sync_copy(z_hbm.at[pl.ds(sid * RPT, RPT)],
                        acc.at[pl.ds(sid * RPT, RPT)])
        pltpu.make_async_copy(idx_hbm.at[pl.ds(base, CH)], ib, gsem[0]).wait()
        pltpu.make_async_copy(dlt_hbm.at[pl.ds(base, CH)], db, gsem[1]).wait()

        def rc_body(j, carry):
            for t in range(GS // 16):
                sl = pl.ds(j * GS + t * 16, 16)
                v = ib[sl]
                ib[sl] = lax.shift_right_logical(v, 13)
                cb[j, pl.ds(t * 16, 16)] = lax.bitwise_and(v, D - 1)
            return carry

        lax.fori_loop(0, NSUB, rc_body, 0)
        plsc.subcore_barrier()

        def g_start(j, b):
            pltpu.async_copy(xT_hbm.at[ib.at[pl.ds(j * GS, GS)]],
                             rows[b], gsem[b])

        def g_wait(b):
            pltpu.make_async_copy(xT_hbm.at[ib.at[pl.ds(0, GS)]],
                                  rows[b], gsem[b]).wait()

        def s_wait(b):
            pltpu.make_async_copy(rows[b], acc.at[cb.at[0]], ssem[b]).wait()

        for j0 in range(LOOK):
            g_start(j0, j0)

        def body(q, carry):
            for b in range(NBUF):
                j = q * NBUF + b
                bl = (b + LOOK) % NBUF
                @pl.when(jnp.logical_and(j + LOOK >= NBUF, j + LOOK < NSUB))
                def _():
                    s_wait(bl)

                @pl.when(j + LOOK < NSUB)
                def _():
                    g_start(j + LOOK, bl)

                g_wait(b)
                r = rows[b]
                for g in range(GS // 16):
                    dvec = db[pl.ds(j * GS + g * 16, 16)]
                    for l in range(16):
                        i = g * 16 + l
                        sl = pl.ds(0, B)
                        lo, hi = plsc.unpack(
                            r[i, sl], format=plsc.PackFormat.INTERLEAVED)
                        dscal = dvec[l]
                        r[i, sl] = plsc.pack(
                            lo * dscal, hi * dscal,
                            format=plsc.PackFormat.INTERLEAVED)
                pltpu.async_copy(r, acc.at[cb.at[j]], ssem[b], add=True)
            return carry

        lax.fori_loop(0, NSUB // NBUF, body, 0)
        for b in range(NBUF):
            s_wait(b)
        plsc.subcore_barrier()
        pltpu.sync_copy(acc.at[pl.ds(sid * RPT, RPT)],
                        out_hbm.at[pl.ds(cid * D + sid * RPT, RPT)])

    return k(xTb, idx_p, dlt_p, zeros)


def _mm_body(x_ref, w_ref, o_ref):
    o_ref[...] = jnp.dot(x_ref[...], w_ref[...],
                         preferred_element_type=jnp.float32)


def _matmul(x, W):
    grid = (D // BN,)
    return pl.pallas_call(
        _mm_body,
        grid=grid,
        in_specs=[
            pl.BlockSpec((B, M), lambda j: (0, 0)),
            pl.BlockSpec((M, BN), lambda j: (0, j)),
        ],
        out_specs=pl.BlockSpec((B, BN), lambda j: (0, j)),
        out_shape=jax.ShapeDtypeStruct((B, D), jnp.float32),
    )(x, W)


def _add_body(d_ref, p_ref, o_ref):
    p = (p_ref[0].astype(jnp.float32) + p_ref[1].astype(jnp.float32))
    o_ref[...] = d_ref[...] + p.T


def _add_partials(dense, P3):
    grid = (D // BN,)
    return pl.pallas_call(
        _add_body,
        grid=grid,
        in_specs=[
            pl.BlockSpec((B, BN), lambda j: (0, j)),
            pl.BlockSpec((2, BN, B), lambda j: (0, j, 0)),
        ],
        out_specs=pl.BlockSpec((B, BN), lambda j: (0, j)),
        out_shape=jax.ShapeDtypeStruct((B, D), jnp.float32),
    )(dense, P3)


def kernel(W, deltas, x, idx):
    pad = NNZ_PAD - NNZ
    idx_p = jnp.concatenate([idx, jnp.zeros((pad,), jnp.int32)])
    dlt_p = jnp.concatenate([deltas, jnp.zeros((pad,), jnp.float32)])
    xTb = x.T.astype(jnp.bfloat16)
    zeros = jnp.zeros((D, B), jnp.bfloat16)
    P = _sc_sparse_partials(xTb, idx_p, dlt_p, zeros)
    dense = _matmul(x, W)
    P3 = P.reshape(2, D, B)
    return _add_partials(dense, P3)


# speedup vs baseline: 10.1313x; 1.0204x over previous
import functools

import jax
import jax.numpy as jnp
from jax import lax
from jax.experimental import pallas as pl
from jax.experimental.pallas import tpu as pltpu
from jax.experimental.pallas import tpu_sc as plsc

M, D, B = 8192, 8192, 32
NNZ = 671088
NW = 32
GS = 128
CH = -(-(-(-NNZ // NW)) // GS) * GS
NNZ_PAD = NW * CH
NSUB = CH // GS
NBUF = 4
LOOK = 2
BN = 512
RPT = D // 16


def _sc_sparse_partials(xTb, idx_p, dlt_p, zeros):
    mesh = plsc.VectorSubcoreMesh(core_axis_name="c", subcore_axis_name="s")

    @functools.partial(
        pl.kernel,
        mesh=mesh,
        compiler_params=pltpu.CompilerParams(use_tc_tiling_on_sc=False,
                                             needs_layout_passes=False),
        out_type=jax.ShapeDtypeStruct((2 * D, B), jnp.bfloat16),
        scratch_types=[
            pltpu.VMEM((CH,), jnp.int32),
            pltpu.VMEM((NSUB, GS), jnp.int32),
            pltpu.VMEM((CH,), jnp.uint32),
            [pltpu.VMEM((GS, B), jnp.bfloat16) for _ in range(NBUF)],
            [pltpu.SemaphoreType.DMA for _ in range(NBUF)],
            [pltpu.SemaphoreType.DMA for _ in range(NBUF)],
            pltpu.VMEM_SHARED((D, B), jnp.bfloat16),
        ],
    )
    def k(xT_hbm, idx_hbm, dlt_hbm, z_hbm, out_hbm,
          ib, cb, db, rows, gsem, ssem, acc):
        cid = lax.axis_index("c")
        sid = lax.axis_index("s")
        wid = cid * 16 + sid
        base = wid * CH
        pltpu.async_copy(idx_hbm.at[pl.ds(base, CH)], ib, gsem[0])
        pltpu.async_copy(dlt_hbm.at[pl.ds(base, CH)], db, gsem[1])
        pltpu.sync_copy(z_hbm.at[pl.ds(sid * RPT, RPT)],
                        acc.at[pl.ds(sid * RPT, RPT)])
        pltpu.make_async_copy(idx_hbm.at[pl.ds(base, CH)], ib, gsem[0]).wait()
        pltpu.make_async_copy(dlt_hbm.at[pl.ds(base, CH)], db, gsem[1]).wait()

        def rc_body(j, carry):
            for t in range(GS // 16):
                sl = pl.ds(j * GS + t * 16, 16)
                v = ib[sl]
                ib[sl] = lax.shift_right_logical(v, 13)
                cb[j, pl.ds(t * 16, 16)] = lax.bitwise_and(v, D - 1)
            return carry

        lax.fori_loop(0, NSUB, rc_body, 0)
        plsc.subcore_barrier()

        def g_start(j, b):
            pltpu.async_copy(xT_hbm.at[ib.at[pl.ds(j * GS, GS)]],
                             rows[b], gsem[b])

        def g_wait(b):
            pltpu.make_async_copy(xT_hbm.at[ib.at[pl.ds(0, GS)]],
                                  rows[b], gsem[b]).wait()

        def s_wait(b):
            pltpu.make_async_copy(rows[b], acc.at[cb.at[0]], ssem[b]).wait()

        for j0 in range(LOOK):
            g_start(j0, j0)

        def body(q, carry):
            for b in range(NBUF):
                j = q * NBUF + b
                bl = (b + LOOK) % NBUF
                @pl.when(jnp.logical_and(j + LOOK >= NBUF, j + LOOK < NSUB))
                def _():
                    s_wait(bl)

                @pl.when(j + LOOK < NSUB)
                def _():
                    g_start(j + LOOK, bl)

                g_wait(b)
                r = rows[b]
                for g in range(GS // 16):
                    dvec = db[pl.ds(j * GS + g * 16, 16)]
                    for l in range(16):
                        i = g * 16 + l
                        sl = pl.ds(0, B)
                        spl = jnp.full((16,), dvec[l], jnp.uint32)
                        dsb = plsc.bitcast(spl, jnp.bfloat16)
                        r[i, sl] = r[i, sl] * dsb
                pltpu.async_copy(r, acc.at[cb.at[j]], ssem[b], add=True)
            return carry

        lax.fori_loop(0, NSUB // NBUF, body, 0)
        for b in range(NBUF):
            s_wait(b)
        plsc.subcore_barrier()
        pltpu.sync_copy(acc.at[pl.ds(sid * RPT, RPT)],
                        out_hbm.at[pl.ds(cid * D + sid * RPT, RPT)])

    return k(xTb, idx_p, dlt_p, zeros)


def _mm_body(x_ref, w_ref, o_ref):
    o_ref[...] = jnp.dot(x_ref[...], w_ref[...],
                         preferred_element_type=jnp.float32)


def _matmul(x, W):
    grid = (D // BN,)
    return pl.pallas_call(
        _mm_body,
        grid=grid,
        in_specs=[
            pl.BlockSpec((B, M), lambda j: (0, 0)),
            pl.BlockSpec((M, BN), lambda j: (0, j)),
        ],
        out_specs=pl.BlockSpec((B, BN), lambda j: (0, j)),
        out_shape=jax.ShapeDtypeStruct((B, D), jnp.float32),
    )(x, W)


def _add_body(d_ref, p_ref, o_ref):
    p = (p_ref[0].astype(jnp.float32) + p_ref[1].astype(jnp.float32))
    o_ref[...] = d_ref[...] + p.T


def _add_partials(dense, P3):
    grid = (D // BN,)
    return pl.pallas_call(
        _add_body,
        grid=grid,
        in_specs=[
            pl.BlockSpec((B, BN), lambda j: (0, j)),
            pl.BlockSpec((2, BN, B), lambda j: (0, j, 0)),
        ],
        out_specs=pl.BlockSpec((B, BN), lambda j: (0, j)),
        out_shape=jax.ShapeDtypeStruct((B, D), jnp.float32),
    )(dense, P3)


def kernel(W, deltas, x, idx):
    pad = NNZ_PAD - NNZ
    pad_idx = (jnp.arange(pad, dtype=jnp.int32) % M) * D
    idx_p = jnp.concatenate([idx, pad_idx])
    dlt_p = jnp.concatenate([deltas, jnp.zeros((pad,), jnp.float32)])
    d16 = jax.lax.bitcast_convert_type(
        dlt_p.astype(jnp.bfloat16), jnp.uint16).astype(jnp.uint32)
    dlt_u = d16 | (d16 << 16)
    xTb = x.T.astype(jnp.bfloat16)
    zeros = jnp.zeros((D, B), jnp.bfloat16)
    P = _sc_sparse_partials(xTb, idx_p, dlt_u, zeros)
    dense = _matmul(x, W)
    P3 = P.reshape(2, D, B)
    return _add_partials(dense, P3)


# speedup vs baseline: 13.3692x; 1.3196x over previous
import functools

import jax
import jax.numpy as jnp
from jax import lax
from jax.experimental import pallas as pl
from jax.experimental.pallas import tpu as pltpu
from jax.experimental.pallas import tpu_sc as plsc

M, D, B = 8192, 8192, 32
NNZ = 671088
NW = 32
GS = 128
CH = -(-(-(-NNZ // NW)) // GS) * GS
NNZ_PAD = NW * CH
NSUB = CH // GS
NBUF = 4
LOOK = 2
BN = 512
RPT = D // 16


def _sc_sparse_partials(xTb, idx_p, dlt_p, zeros):
    mesh = plsc.VectorSubcoreMesh(core_axis_name="c", subcore_axis_name="s")

    @functools.partial(
        pl.kernel,
        mesh=mesh,
        compiler_params=pltpu.CompilerParams(use_tc_tiling_on_sc=False,
                                             needs_layout_passes=False),
        out_type=jax.ShapeDtypeStruct((2, D, B), jnp.bfloat16),
        scratch_types=[
            pltpu.VMEM((CH,), jnp.int32),
            pltpu.VMEM((NSUB, GS), jnp.int32),
            pltpu.VMEM((CH,), jnp.uint32),
            [pltpu.VMEM((GS, B), jnp.bfloat16) for _ in range(NBUF)],
            [pltpu.SemaphoreType.DMA for _ in range(NBUF)],
            [pltpu.SemaphoreType.DMA for _ in range(NBUF)],
            pltpu.VMEM_SHARED((D, B), jnp.bfloat16),
            pltpu.VMEM_SHARED((D, B), jnp.bfloat16),
        ],
    )
    def k(xT_hbm, idx_hbm, dlt_hbm, z_hbm, out_hbm,
          ib, cb, db, rows, gsem, ssem, acc, xsh):
        cid = lax.axis_index("c")
        sid = lax.axis_index("s")
        wid = cid * 16 + sid
        base = wid * CH
        pltpu.async_copy(idx_hbm.at[pl.ds(base, CH)], ib, gsem[0])
        pltpu.async_copy(dlt_hbm.at[pl.ds(base, CH)], db, gsem[1])
        pltpu.sync_copy(xT_hbm.at[pl.ds(sid * RPT, RPT)],
                        xsh.at[pl.ds(sid * RPT, RPT)])
        pltpu.sync_copy(z_hbm.at[pl.ds(sid * RPT, RPT)],
                        acc.at[pl.ds(sid * RPT, RPT)])
        pltpu.make_async_copy(idx_hbm.at[pl.ds(base, CH)], ib, gsem[0]).wait()
        pltpu.make_async_copy(dlt_hbm.at[pl.ds(base, CH)], db, gsem[1]).wait()

        def rc_body(j, carry):
            for t in range(GS // 16):
                sl = pl.ds(j * GS + t * 16, 16)
                v = ib[sl]
                ib[sl] = lax.shift_right_logical(v, 13)
                cb[j, pl.ds(t * 16, 16)] = lax.bitwise_and(v, D - 1)
            return carry

        lax.fori_loop(0, NSUB, rc_body, 0)
        plsc.subcore_barrier()

        def g_start(j, b):
            pltpu.async_copy(xsh.at[ib.at[pl.ds(j * GS, GS)]],
                             rows[b], gsem[b])

        def g_wait(b):
            pltpu.make_async_copy(xsh.at[ib.at[pl.ds(0, GS)]],
                                  rows[b], gsem[b]).wait()

        def s_wait(b):
            pltpu.make_async_copy(rows[b], acc.at[cb.at[0]], ssem[b]).wait()

        for j0 in range(LOOK):
            g_start(j0, j0)

        def body(q, carry):
            for b in range(NBUF):
                j = q * NBUF + b
                bl = (b + LOOK) % NBUF
                @pl.when(jnp.logical_and(j + LOOK >= NBUF, j + LOOK < NSUB))
                def _():
                    s_wait(bl)

                @pl.when(j + LOOK < NSUB)
                def _():
                    g_start(j + LOOK, bl)

                g_wait(b)
                r = rows[b]
                for g in range(GS // 16):
                    dvec = db[pl.ds(j * GS + g * 16, 16)]
                    for l in range(16):
                        i = g * 16 + l
                        sl = pl.ds(0, B)
                        spl = jnp.full((16,), dvec[l], jnp.uint32)
                        dsb = plsc.bitcast(spl, jnp.bfloat16)
                        r[i, sl] = r[i, sl] * dsb
                pltpu.async_copy(r, acc.at[cb.at[j]], ssem[b], add=True)
            return carry

        lax.fori_loop(0, NSUB // NBUF, body, 0)
        for b in range(NBUF):
            s_wait(b)
        plsc.subcore_barrier()
        pltpu.sync_copy(acc.at[pl.ds(sid * RPT, RPT)],
                        out_hbm.at[cid, pl.ds(sid * RPT, RPT)])

    return k(xTb, idx_p, dlt_p, zeros)


def _mm_body(x_ref, w_ref, o_ref):
    o_ref[...] = jnp.dot(x_ref[...], w_ref[...],
                         preferred_element_type=jnp.float32)


def _matmul(x, W):
    grid = (D // BN,)
    return pl.pallas_call(
        _mm_body,
        grid=grid,
        in_specs=[
            pl.BlockSpec((B, M), lambda j: (0, 0)),
            pl.BlockSpec((M, BN), lambda j: (0, j)),
        ],
        out_specs=pl.BlockSpec((B, BN), lambda j: (0, j)),
        out_shape=jax.ShapeDtypeStruct((B, D), jnp.float32),
    )(x, W)


BA = 2048


def _add_body(d_ref, p_ref, o_ref):
    p = (p_ref[0] + p_ref[1]).astype(jnp.float32)
    eye = (lax.broadcasted_iota(jnp.int32, (B, B), 0)
           == lax.broadcasted_iota(jnp.int32, (B, B), 1)).astype(jnp.float32)
    pt = lax.dot_general(eye, p, (((1,), (1,)), ((), ())),
                         preferred_element_type=jnp.float32)
    o_ref[...] = d_ref[...] + pt


def _add_partials(dense, P3):
    grid = (D // BA,)
    return pl.pallas_call(
        _add_body,
        grid=grid,
        in_specs=[
            pl.BlockSpec((B, BA), lambda j: (0, j)),
            pl.BlockSpec((2, BA, B), lambda j: (0, j, 0)),
        ],
        out_specs=pl.BlockSpec((B, BA), lambda j: (0, j)),
        out_shape=jax.ShapeDtypeStruct((B, D), jnp.float32),
    )(dense, P3)


def kernel(W, deltas, x, idx):
    pad = NNZ_PAD - NNZ
    pad_idx = (jnp.arange(pad, dtype=jnp.int32) % M) * D
    idx_p = jnp.concatenate([idx, pad_idx])
    dlt_p = jnp.concatenate([deltas, jnp.zeros((pad,), jnp.float32)])
    d16 = jax.lax.bitcast_convert_type(
        dlt_p.astype(jnp.bfloat16), jnp.uint16).astype(jnp.uint32)
    dlt_u = d16 | (d16 << 16)
    xTb = x.T.astype(jnp.bfloat16)
    zeros = jnp.zeros((D, B), jnp.bfloat16)
    P3 = _sc_sparse_partials(xTb, idx_p, dlt_u, zeros)
    dense = _matmul(x, W)
    return _add_partials(dense, P3)


# speedup vs baseline: 13.3919x; 1.0017x over previous
import functools

import jax
import jax.numpy as jnp
from jax import lax
from jax.experimental import pallas as pl
from jax.experimental.pallas import tpu as pltpu
from jax.experimental.pallas import tpu_sc as plsc

M, D, B = 8192, 8192, 32
NNZ = 671088
NW = 32
GS = 128
CH = -(-(-(-NNZ // NW)) // GS) * GS
NSUB = CH // GS
OV = NW * CH - NNZ
NBUF = 4
LOOK = 2
BN = 512
RPT = D // 16


def _sc_sparse_partials(xTb, idx_p, dlt_p, zeros):
    mesh = plsc.VectorSubcoreMesh(core_axis_name="c", subcore_axis_name="s")

    @functools.partial(
        pl.kernel,
        mesh=mesh,
        compiler_params=pltpu.CompilerParams(use_tc_tiling_on_sc=False,
                                             needs_layout_passes=False),
        out_type=jax.ShapeDtypeStruct((2, D, B), jnp.bfloat16),
        scratch_types=[
            pltpu.VMEM((CH,), jnp.int32),
            pltpu.VMEM((NSUB, GS), jnp.int32),
            pltpu.VMEM((CH,), jnp.uint32),
            [pltpu.VMEM((GS, B), jnp.bfloat16) for _ in range(NBUF)],
            [pltpu.SemaphoreType.DMA for _ in range(NBUF)],
            [pltpu.SemaphoreType.DMA for _ in range(NBUF)],
            pltpu.VMEM_SHARED((D, B), jnp.bfloat16),
            pltpu.VMEM_SHARED((D, B), jnp.bfloat16),
        ],
    )
    def k(xT_hbm, idx_hbm, dlt_hbm, z_hbm, out_hbm,
          ib, cb, db, rows, gsem, ssem, acc, xsh):
        cid = lax.axis_index("c")
        sid = lax.axis_index("s")
        wid = cid * 16 + sid
        base = jnp.minimum(wid * CH, NNZ - CH)
        pltpu.async_copy(idx_hbm.at[pl.ds(base, CH)], ib, gsem[0])
        pltpu.async_copy(dlt_hbm.at[pl.ds(base, CH)], db, gsem[1])
        pltpu.sync_copy(xT_hbm.at[pl.ds(sid * RPT, RPT)],
                        xsh.at[pl.ds(sid * RPT, RPT)])
        pltpu.sync_copy(z_hbm.at[pl.ds(sid * RPT, RPT)],
                        acc.at[pl.ds(sid * RPT, RPT)])
        pltpu.make_async_copy(idx_hbm.at[pl.ds(base, CH)], ib, gsem[0]).wait()
        pltpu.make_async_copy(dlt_hbm.at[pl.ds(base, CH)], db, gsem[1]).wait()

        @pl.when(wid == NW - 1)
        def _():
            zv = jnp.zeros((16,), jnp.uint32)
            for t in range(OV // 16):
                db[pl.ds(t * 16, 16)] = zv

        def rc_body(j, carry):
            for t in range(GS // 16):
                sl = pl.ds(j * GS + t * 16, 16)
                v = ib[sl]
                ib[sl] = lax.shift_right_logical(v, 13)
                cb[j, pl.ds(t * 16, 16)] = lax.bitwise_and(v, D - 1)
            return carry

        lax.fori_loop(0, NSUB, rc_body, 0)
        plsc.subcore_barrier()

        def g_start(j, b):
            pltpu.async_copy(xsh.at[ib.at[pl.ds(j * GS, GS)]],
                             rows[b], gsem[b])

        def g_wait(b):
            pltpu.make_async_copy(xsh.at[ib.at[pl.ds(0, GS)]],
                                  rows[b], gsem[b]).wait()

        def s_wait(b):
            pltpu.make_async_copy(rows[b], acc.at[cb.at[0]], ssem[b]).wait()

        for j0 in range(LOOK):
            g_start(j0, j0)

        def body(q, carry):
            for b in range(NBUF):
                j = q * NBUF + b
                bl = (b + LOOK) % NBUF
                @pl.when(jnp.logical_and(j + LOOK >= NBUF, j + LOOK < NSUB))
                def _():
                    s_wait(bl)

                @pl.when(j + LOOK < NSUB)
                def _():
                    g_start(j + LOOK, bl)

                g_wait(b)
                r = rows[b]
                for g in range(GS // 16):
                    dvec = db[pl.ds(j * GS + g * 16, 16)]
                    for l in range(16):
                        i = g * 16 + l
                        sl = pl.ds(0, B)
                        spl = jnp.full((16,), dvec[l], jnp.uint32)
                        dsb = plsc.bitcast(spl, jnp.bfloat16)
                        r[i, sl] = r[i, sl] * dsb
                pltpu.async_copy(r, acc.at[cb.at[j]], ssem[b], add=True)
            return carry

        lax.fori_loop(0, NSUB // NBUF, body, 0)
        for b in range(NBUF):
            s_wait(b)
        plsc.subcore_barrier()
        pltpu.sync_copy(acc.at[pl.ds(sid * RPT, RPT)],
                        out_hbm.at[cid, pl.ds(sid * RPT, RPT)])

    return k(xTb, idx_p, dlt_p, zeros)


def _mm_body(x_ref, w_ref, o_ref):
    o_ref[...] = jnp.dot(x_ref[...], w_ref[...],
                         preferred_element_type=jnp.float32)


def _matmul(x, W):
    grid = (D // BN,)
    return pl.pallas_call(
        _mm_body,
        grid=grid,
        in_specs=[
            pl.BlockSpec((B, M), lambda j: (0, 0)),
            pl.BlockSpec((M, BN), lambda j: (0, j)),
        ],
        out_specs=pl.BlockSpec((B, BN), lambda j: (0, j)),
        out_shape=jax.ShapeDtypeStruct((B, D), jnp.float32),
    )(x, W)


BA = 2048


def _add_body(d_ref, p_ref, o_ref):
    p = (p_ref[0] + p_ref[1]).astype(jnp.float32)
    eye = (lax.broadcasted_iota(jnp.int32, (B, B), 0)
           == lax.broadcasted_iota(jnp.int32, (B, B), 1)).astype(jnp.float32)
    pt = lax.dot_general(eye, p, (((1,), (1,)), ((), ())),
                         preferred_element_type=jnp.float32)
    o_ref[...] = d_ref[...] + pt


def _add_partials(dense, P3):
    grid = (D // BA,)
    return pl.pallas_call(
        _add_body,
        grid=grid,
        in_specs=[
            pl.BlockSpec((B, BA), lambda j: (0, j)),
            pl.BlockSpec((2, BA, B), lambda j: (0, j, 0)),
        ],
        out_specs=pl.BlockSpec((B, BA), lambda j: (0, j)),
        out_shape=jax.ShapeDtypeStruct((B, D), jnp.float32),
        input_output_aliases={0: 0},
    )(dense, P3)


def kernel(W, deltas, x, idx):
    d16 = jax.lax.bitcast_convert_type(
        deltas.astype(jnp.bfloat16), jnp.uint16).astype(jnp.uint32)
    dlt_u = d16 | (d16 << 16)
    xTb = x.T.astype(jnp.bfloat16)
    zeros = jnp.zeros((D, B), jnp.bfloat16)
    P3 = _sc_sparse_partials(xTb, idx, dlt_u, zeros)
    dense = _matmul(x, W)
    return _add_partials(dense, P3)


# speedup vs baseline: 14.0352x; 1.0480x over previous
import functools

import jax
import jax.numpy as jnp
from jax import lax
from jax.experimental import pallas as pl
from jax.experimental.pallas import tpu as pltpu
from jax.experimental.pallas import tpu_sc as plsc

M, D, B = 8192, 8192, 32
NNZ = 671088
NW = 32
GS = 128
CH = -(-(-(-NNZ // NW)) // GS) * GS
NSUB = CH // GS
OV = NW * CH - NNZ
NBUF = 4
LOOK = 2
BN = 512
RPT = D // 16


def _sc_sparse_partials(xTb, idx_p, dlt_p):
    mesh = plsc.VectorSubcoreMesh(core_axis_name="c", subcore_axis_name="s")

    @functools.partial(
        pl.kernel,
        mesh=mesh,
        compiler_params=pltpu.CompilerParams(use_tc_tiling_on_sc=False,
                                             needs_layout_passes=False),
        out_type=jax.ShapeDtypeStruct((2, D, B), jnp.bfloat16),
        scratch_types=[
            pltpu.VMEM((CH,), jnp.int32),
            pltpu.VMEM((NSUB, GS), jnp.int32),
            pltpu.VMEM((CH,), jnp.float32),
            [pltpu.VMEM((GS, B), jnp.bfloat16) for _ in range(NBUF)],
            [pltpu.SemaphoreType.DMA for _ in range(NBUF)],
            [pltpu.SemaphoreType.DMA for _ in range(NBUF)],
            pltpu.VMEM_SHARED((D, B), jnp.bfloat16),
            pltpu.VMEM_SHARED((D, B), jnp.bfloat16),
        ],
    )
    def k(xT_hbm, idx_hbm, dlt_hbm, out_hbm,
          ib, cb, db, rows, gsem, ssem, acc, xsh):
        cid = lax.axis_index("c")
        sid = lax.axis_index("s")
        wid = cid * 16 + sid
        base = jnp.minimum(wid * CH, NNZ - CH)
        pltpu.async_copy(idx_hbm.at[pl.ds(base, CH)], ib, gsem[0])
        pltpu.async_copy(dlt_hbm.at[pl.ds(base, CH)], db, gsem[1])
        pltpu.sync_copy(xT_hbm.at[pl.ds(sid * RPT, RPT)],
                        xsh.at[pl.ds(sid * RPT, RPT)])
        zv = jnp.zeros((B,), jnp.bfloat16)
        for t in range(GS):
            rows[0][t, pl.ds(0, B)] = zv
        for t in range(RPT // GS):
            pltpu.sync_copy(rows[0],
                            acc.at[pl.ds(sid * RPT + t * GS, GS)])
        pltpu.make_async_copy(idx_hbm.at[pl.ds(base, CH)], ib, gsem[0]).wait()
        pltpu.make_async_copy(dlt_hbm.at[pl.ds(base, CH)], db, gsem[1]).wait()

        @pl.when(wid == NW - 1)
        def _():
            zf = jnp.zeros((16,), jnp.float32)
            for t in range(OV // 16):
                db[pl.ds(t * 16, 16)] = zf

        def rc_body(j, carry):
            for t in range(GS // 16):
                sl = pl.ds(j * GS + t * 16, 16)
                v = ib[sl]
                ib[sl] = lax.shift_right_logical(v, 13)
                cb[j, pl.ds(t * 16, 16)] = lax.bitwise_and(v, D - 1)
            return carry

        lax.fori_loop(0, NSUB, rc_body, 0)
        plsc.subcore_barrier()

        def g_start(j, b):
            pltpu.async_copy(xsh.at[ib.at[pl.ds(j * GS, GS)]],
                             rows[b], gsem[b])

        def g_wait(b):
            pltpu.make_async_copy(xsh.at[ib.at[pl.ds(0, GS)]],
                                  rows[b], gsem[b]).wait()

        def s_wait(b):
            pltpu.make_async_copy(rows[b], acc.at[cb.at[0]], ssem[b]).wait()

        for j0 in range(LOOK):
            g_start(j0, j0)

        def body(q, carry):
            for b in range(NBUF):
                j = q * NBUF + b
                bl = (b + LOOK) % NBUF
                @pl.when(jnp.logical_and(j + LOOK >= NBUF, j + LOOK < NSUB))
                def _():
                    s_wait(bl)

                @pl.when(j + LOOK < NSUB)
                def _():
                    g_start(j + LOOK, bl)

                g_wait(b)
                r = rows[b]
                for g in range(GS // 16):
                    dvec = db[pl.ds(j * GS + g * 16, 16)]
                    dup = plsc.pack(dvec, dvec,
                                    format=plsc.PackFormat.INTERLEAVED)
                    du = plsc.bitcast(dup, jnp.uint32)
                    for l in range(16):
                        i = g * 16 + l
                        sl = pl.ds(0, B)
                        spl = jnp.full((16,), du[l], jnp.uint32)
                        dsb = plsc.bitcast(spl, jnp.bfloat16)
                        r[i, sl] = r[i, sl] * dsb
                pltpu.async_copy(r, acc.at[cb.at[j]], ssem[b], add=True)
            return carry

        lax.fori_loop(0, NSUB // NBUF, body, 0)
        for b in range(NBUF):
            s_wait(b)
        plsc.subcore_barrier()
        pltpu.sync_copy(acc.at[pl.ds(sid * RPT, RPT)],
                        out_hbm.at[cid, pl.ds(sid * RPT, RPT)])

    return k(xTb, idx_p, dlt_p)


def _mm_body(x_ref, w_ref, o_ref):
    o_ref[...] = jnp.dot(x_ref[...], w_ref[...],
                         preferred_element_type=jnp.float32)


def _matmul(x, W):
    grid = (D // BN,)
    return pl.pallas_call(
        _mm_body,
        grid=grid,
        in_specs=[
            pl.BlockSpec((B, M), lambda j: (0, 0)),
            pl.BlockSpec((M, BN), lambda j: (0, j)),
        ],
        out_specs=pl.BlockSpec((B, BN), lambda j: (0, j)),
        out_shape=jax.ShapeDtypeStruct((B, D), jnp.float32),
    )(x, W)


BA = 2048


def _add_body(d_ref, p_ref, o_ref):
    p = (p_ref[0] + p_ref[1]).astype(jnp.float32)
    eye = (lax.broadcasted_iota(jnp.int32, (B, B), 0)
           == lax.broadcasted_iota(jnp.int32, (B, B), 1)).astype(jnp.float32)
    pt = lax.dot_general(eye, p, (((1,), (1,)), ((), ())),
                         preferred_element_type=jnp.float32)
    o_ref[...] = d_ref[...] + pt


def _add_partials(dense, P3):
    grid = (D // BA,)
    return pl.pallas_call(
        _add_body,
        grid=grid,
        in_specs=[
            pl.BlockSpec((B, BA), lambda j: (0, j)),
            pl.BlockSpec((2, BA, B), lambda j: (0, j, 0)),
        ],
        out_specs=pl.BlockSpec((B, BA), lambda j: (0, j)),
        out_shape=jax.ShapeDtypeStruct((B, D), jnp.float32),
        input_output_aliases={0: 0},
    )(dense, P3)


def kernel(W, deltas, x, idx):
    xTb = x.T.astype(jnp.bfloat16)
    P3 = _sc_sparse_partials(xTb, idx, deltas)
    dense = _matmul(x, W)
    return _add_partials(dense, P3)


# speedup vs baseline: 14.0381x; 1.0002x over previous
import functools

import jax
import jax.numpy as jnp
from jax import lax
from jax.experimental import pallas as pl
from jax.experimental.pallas import tpu as pltpu
from jax.experimental.pallas import tpu_sc as plsc

M, D, B = 8192, 8192, 32
NNZ = 671088
NW = 32
GS = 128
CH = -(-(-(-NNZ // NW)) // GS) * GS
NSUB = CH // GS
OV = NW * CH - NNZ
NBUF = 4
LOOK = 2
BN = 512
RPT = D // 16


def _sc_sparse_partials(xTb, idx_p, dlt_p):
    mesh = plsc.VectorSubcoreMesh(core_axis_name="c", subcore_axis_name="s")

    @functools.partial(
        pl.kernel,
        mesh=mesh,
        compiler_params=pltpu.CompilerParams(use_tc_tiling_on_sc=False,
                                             needs_layout_passes=False),
        out_type=jax.ShapeDtypeStruct((2, D, B), jnp.bfloat16),
        scratch_types=[
            pltpu.VMEM((CH,), jnp.int32),
            pltpu.VMEM((NSUB, GS), jnp.int32),
            pltpu.VMEM((CH,), jnp.float32),
            [pltpu.VMEM((GS, B), jnp.bfloat16) for _ in range(NBUF)],
            [pltpu.SemaphoreType.DMA for _ in range(NBUF)],
            [pltpu.SemaphoreType.DMA for _ in range(NBUF)],
            pltpu.VMEM_SHARED((D, B), jnp.bfloat16),
            pltpu.VMEM_SHARED((D, B), jnp.bfloat16),
        ],
    )
    def k(xT_hbm, idx_hbm, dlt_hbm, out_hbm,
          ib, cb, db, rows, gsem, ssem, acc, xsh):
        cid = lax.axis_index("c")
        sid = lax.axis_index("s")
        wid = cid * 16 + sid
        base = jnp.minimum(wid * CH, NNZ - CH)
        pltpu.async_copy(idx_hbm.at[pl.ds(base, CH)], ib, gsem[0])
        pltpu.async_copy(dlt_hbm.at[pl.ds(base, CH)], db, gsem[1])
        pltpu.sync_copy(xT_hbm.at[pl.ds(sid * RPT, RPT)],
                        xsh.at[pl.ds(sid * RPT, RPT)])
        zv = jnp.zeros((B,), jnp.bfloat16)
        for t in range(GS):
            rows[0][t, pl.ds(0, B)] = zv
        for t in range(RPT // GS):
            pltpu.sync_copy(rows[0],
                            acc.at[pl.ds(sid * RPT + t * GS, GS)])
        pltpu.make_async_copy(idx_hbm.at[pl.ds(base, CH)], ib, gsem[0]).wait()
        pltpu.make_async_copy(dlt_hbm.at[pl.ds(base, CH)], db, gsem[1]).wait()

        @pl.when(wid == NW - 1)
        def _():
            zf = jnp.zeros((16,), jnp.float32)
            for t in range(OV // 16):
                db[pl.ds(t * 16, 16)] = zf

        def rc_body(j, carry):
            for t in range(GS // 16):
                sl = pl.ds(j * GS + t * 16, 16)
                v = ib[sl]
                ib[sl] = lax.shift_right_logical(v, 13)
                cb[j, pl.ds(t * 16, 16)] = lax.bitwise_and(v, D - 1)
            return carry

        lax.fori_loop(0, NSUB, rc_body, 0)
        plsc.subcore_barrier()

        def g_start(j, b):
            pltpu.async_copy(xsh.at[ib.at[pl.ds(j * GS, GS)]],
                             rows[b], gsem[b])

        def g_wait(b):
            pltpu.make_async_copy(xsh.at[ib.at[pl.ds(0, GS)]],
                                  rows[b], gsem[b]).wait()

        def s_wait(b):
            pltpu.make_async_copy(rows[b], acc.at[cb.at[0]], ssem[b]).wait()

        for j0 in range(LOOK):
            g_start(j0, j0)

        def body(q, carry):
            for b in range(NBUF):
                j = q * NBUF + b
                bl = (b + LOOK) % NBUF
                @pl.when(jnp.logical_and(j + LOOK >= NBUF, j + LOOK < NSUB))
                def _():
                    s_wait(bl)

                @pl.when(j + LOOK < NSUB)
                def _():
                    g_start(j + LOOK, bl)

                g_wait(b)
                r = rows[b]
                for g in range(GS // 16):
                    dvec = db[pl.ds(j * GS + g * 16, 16)]
                    dup = plsc.pack(dvec, dvec,
                                    format=plsc.PackFormat.INTERLEAVED)
                    du = plsc.bitcast(dup, jnp.uint32)
                    for l in range(16):
                        i = g * 16 + l
                        sl = pl.ds(0, B)
                        spl = jnp.full((16,), du[l], jnp.uint32)
                        dsb = plsc.bitcast(spl, jnp.bfloat16)
                        r[i, sl] = r[i, sl] * dsb
                pltpu.async_copy(r, acc.at[cb.at[j]], ssem[b], add=True)
            return carry

        lax.fori_loop(0, NSUB // NBUF, body, 0)
        for b in range(NBUF):
            s_wait(b)
        plsc.subcore_barrier()
        pltpu.sync_copy(acc.at[pl.ds(sid * RPT, RPT)],
                        out_hbm.at[cid, pl.ds(sid * RPT, RPT)])

    return k(xTb, idx_p, dlt_p)


def _mm_body(x_ref, w_ref, o_ref):
    o_ref[...] = jnp.dot(x_ref[...], w_ref[...],
                         preferred_element_type=jnp.float32)


def _matmul(x, W):
    grid = (D // BN,)
    return pl.pallas_call(
        _mm_body,
        grid=grid,
        in_specs=[
            pl.BlockSpec((B, M), lambda j: (0, 0)),
            pl.BlockSpec((M, BN), lambda j: (0, j)),
        ],
        out_specs=pl.BlockSpec((B, BN), lambda j: (0, j)),
        out_shape=jax.ShapeDtypeStruct((B, D), jnp.float32),
    )(x, W)


def _add_body(d_ref, p_hbm, o_ref, pv, sem):
    pltpu.make_async_copy(p_hbm, pv, sem).start()
    pltpu.make_async_copy(p_hbm, pv, sem).wait()
    p = (pv[0] + pv[1]).astype(jnp.float32)
    eye = (lax.broadcasted_iota(jnp.int32, (B, B), 0)
           == lax.broadcasted_iota(jnp.int32, (B, B), 1)).astype(jnp.float32)
    pt = lax.dot_general(eye, p, (((1,), (1,)), ((), ())),
                         preferred_element_type=jnp.float32)
    o_ref[...] = d_ref[...] + pt


def _add_partials(dense, P3):
    return pl.pallas_call(
        _add_body,
        in_specs=[
            pl.BlockSpec(memory_space=pltpu.VMEM),
            pl.BlockSpec(memory_space=pl.ANY),
        ],
        out_specs=pl.BlockSpec(memory_space=pltpu.VMEM),
        out_shape=jax.ShapeDtypeStruct((B, D), jnp.float32),
        scratch_shapes=[pltpu.VMEM((2, D, B), jnp.bfloat16),
                        pltpu.SemaphoreType.DMA],
        input_output_aliases={0: 0},
    )(dense, P3)


def kernel(W, deltas, x, idx):
    xTb = x.T.astype(jnp.bfloat16)
    P3 = _sc_sparse_partials(xTb, idx, deltas)
    dense = _matmul(x, W)
    return _add_partials(dense, P3)
